# baseline (speedup 1.0000x reference)
import functools

import jax
import jax.numpy as jnp
from jax import lax
from jax.experimental import pallas as pl
from jax.experimental.pallas import tpu as pltpu
from jax.experimental.pallas import tpu_sc as plsc

EPS = 1e-5

NC = 2
NS = 16
NW = NC * NS

N = 10000
E = 320000
H = 128
ADJ = 16
OUT = 128
VOCABP = 32

NP = 10240
RPT = NP // NS
NSH = 2
ES = E // NSH
EW = ES // NW
C = 40
K = EW // C

BN = 1024
BE = 3200


def _node_init_body(idx_ref, emb_ref, w_ref, b_ref, g_ref, beta_ref, o_ref):
    idx = idx_ref[0]
    oht = (idx == lax.broadcasted_iota(jnp.int32, (VOCABP, BN), 0))
    e = lax.dot_general(oht.astype(jnp.float32), emb_ref[...],
                        (((0,), (0,)), ((), ())))
    e = jnp.maximum(e, 0.0)
    h = e @ w_ref[...] + b_ref[...]
    mu = jnp.mean(h, axis=-1, keepdims=True)
    var = jnp.mean((h - mu) * (h - mu), axis=-1, keepdims=True)
    o_ref[...] = (h - mu) * lax.rsqrt(var + EPS) * g_ref[...] + beta_ref[...]


def _bdot(a, b):
    return lax.dot_general(a.astype(jnp.bfloat16), b.astype(jnp.bfloat16),
                           (((1,), (0,)), ((), ())),
                           preferred_element_type=jnp.float32)


def _edge_mlp(xi, xj, ea, w1_ref, b1_ref, w2_ref, b2_ref):
    h = (_bdot(xi, w1_ref[:H, :]) + _bdot(xj - xi, w1_ref[H:2 * H, :])
         + _bdot(ea, w1_ref[2 * H:, :]) + b1_ref[...])
    h = jnp.maximum(h, 0.0)
    return _bdot(h, w2_ref[...]) + b2_ref[...]


def _edge_first_body(attr_ref, xi_ref, xj_ref,
                     ew1_ref, eb1_ref, ew2_ref, eb2_ref, eg_ref, ebeta_ref,
                     w1_ref, b1_ref, w2_ref, b2_ref, ge_ref, be_ref,
                     m_ref, eo_ref):
    t = jnp.maximum(attr_ref[...] @ ew1_ref[...] + eb1_ref[...], 0.0)
    t = t @ ew2_ref[...] + eb2_ref[...]
    mu = jnp.mean(t, axis=-1, keepdims=True)
    var = jnp.mean((t - mu) * (t - mu), axis=-1, keepdims=True)
    ea = (t - mu) * lax.rsqrt(var + EPS) * eg_ref[...] + ebeta_ref[...]
    m = _edge_mlp(xi_ref[...], xj_ref[...], ea, w1_ref, b1_ref, w2_ref, b2_ref)
    m_ref[...] = m
    eo_ref[...] = jnp.maximum(ea + m * ge_ref[...] + be_ref[...], 0.0)


def _edge_mid_body(ea_ref, xi_ref, xj_ref,
                   w1_ref, b1_ref, w2_ref, b2_ref, ge_ref, be_ref,
                   m_ref, eo_ref):
    ea = ea_ref[...]
    m = _edge_mlp(xi_ref[...], xj_ref[...], ea, w1_ref, b1_ref, w2_ref, b2_ref)
    m_ref[...] = m
    eo_ref[...] = jnp.maximum(ea + m * ge_ref[...] + be_ref[...], 0.0)


def _edge_last_body(ea_ref, xi_ref, xj_ref,
                    w1_ref, b1_ref, w2_ref, b2_ref,
                    m_ref):
    m_ref[...] = _edge_mlp(xi_ref[...], xj_ref[...], ea_ref[...],
                           w1_ref, b1_ref, w2_ref, b2_ref)


def _node_mid_body(x_ref, acca_ref, accb_ref, g_ref, b_ref, o_ref):
    s = (acca_ref[0] + acca_ref[1]) + (accb_ref[0] + accb_ref[1])
    o_ref[...] = jnp.maximum(x_ref[...] + s * g_ref[...] + b_ref[...], 0.0)


def _node_last_body(x_ref, acca_ref, accb_ref, g_ref, b_ref, w_ref, ob_ref,
                    o_ref):
    s = (acca_ref[0] + acca_ref[1]) + (accb_ref[0] + accb_ref[1])
    xn = x_ref[...] + s * g_ref[...] + b_ref[...]
    o_ref[...] = xn @ w_ref[...] + ob_ref[...]


def _row(n):
    return pl.BlockSpec((1, n), lambda i: (0, 0))


def _full(a, b):
    return pl.BlockSpec((a, b), lambda i: (0, 0))


def _tc_node_init(xpad3, embp, w, b, g, beta):
    return pl.pallas_call(
        _node_init_body,
        grid=(NP // BN,),
        in_specs=[
            pl.BlockSpec((1, 1, BN), lambda i: (i, 0, 0)),
            _full(VOCABP, H), _full(H, H), _row(H), _row(H), _row(H),
        ],
        out_specs=pl.BlockSpec((BN, H), lambda i: (i, 0)),
        out_shape=jax.ShapeDtypeStruct((NP, H), jnp.float32),
    )(xpad3, embp, w, b, g, beta)


def _tc_edge_first(attr, xi, xj, pe, pc, ge, be):
    blk = pl.BlockSpec((BE, H), lambda i: (i, 0))
    return pl.pallas_call(
        _edge_first_body,
        grid=(ES // BE,),
        in_specs=[
            pl.BlockSpec((BE, ADJ), lambda i: (i, 0)), blk, blk,
            _full(ADJ, H), _row(H), _full(H, H), _row(H), _row(H), _row(H),
            _full(3 * H, 2 * H), _row(2 * H), _full(2 * H, H), _row(H),
            _row(H), _row(H),
        ],
        out_specs=(blk, blk),
        out_shape=(jax.ShapeDtypeStruct((ES, H), jnp.float32),
                   jax.ShapeDtypeStruct((ES, H), jnp.float32)),
    )(attr, xi, xj,
      pe["ea_W1"], pe["ea_b1"].reshape(1, H), pe["ea_W2"],
      pe["ea_b2"].reshape(1, H), pe["ea_g"].reshape(1, H),
      pe["ea_beta"].reshape(1, H),
      pc["W1"], pc["b1"].reshape(1, 2 * H), pc["W2"], pc["b2"].reshape(1, H),
      ge.reshape(1, H), be.reshape(1, H))


def _tc_edge_mid(ea, xi, xj, pc, ge, be):
    blk = pl.BlockSpec((BE, H), lambda i: (i, 0))
    return pl.pallas_call(
        _edge_mid_body,
        grid=(ES // BE,),
        in_specs=[
            blk, blk, blk,
            _full(3 * H, 2 * H), _row(2 * H), _full(2 * H, H), _row(H),
            _row(H), _row(H),
        ],
        out_specs=(blk, blk),
        out_shape=(jax.ShapeDtypeStruct((ES, H), jnp.float32),
                   jax.ShapeDtypeStruct((ES, H), jnp.float32)),
    )(ea, xi, xj, pc["W1"], pc["b1"].reshape(1, 2 * H), pc["W2"],
      pc["b2"].reshape(1, H), ge.reshape(1, H), be.reshape(1, H))


def _tc_edge_last(ea, xi, xj, pc):
    blk = pl.BlockSpec((BE, H), lambda i: (i, 0))
    return pl.pallas_call(
        _edge_last_body,
        grid=(ES // BE,),
        in_specs=[
            blk, blk, blk,
            _full(3 * H, 2 * H), _row(2 * H), _full(2 * H, H), _row(H),
        ],
        out_specs=blk,
        out_shape=jax.ShapeDtypeStruct((ES, H), jnp.float32),
    )(ea, xi, xj, pc["W1"], pc["b1"].reshape(1, 2 * H), pc["W2"],
      pc["b2"].reshape(1, H))


def _tc_node_mid(x, acca, accb, gx, bx):
    return pl.pallas_call(
        _node_mid_body,
        grid=(NP // BN,),
        in_specs=[
            pl.BlockSpec((BN, H), lambda i: (i, 0)),
            pl.BlockSpec((2, BN, H), lambda i: (0, i, 0)),
            pl.BlockSpec((2, BN, H), lambda i: (0, i, 0)),
            _row(H), _row(H),
        ],
        out_specs=pl.BlockSpec((BN, H), lambda i: (i, 0)),
        out_shape=jax.ShapeDtypeStruct((NP, H), jnp.float32),
    )(x, acca, accb, gx.reshape(1, H), bx.reshape(1, H))


def _tc_node_last(x, acca, accb, gx, bx, w, ob):
    return pl.pallas_call(
        _node_last_body,
        grid=(NP // BN,),
        in_specs=[
            pl.BlockSpec((BN, H), lambda i: (i, 0)),
            pl.BlockSpec((2, BN, H), lambda i: (0, i, 0)),
            pl.BlockSpec((2, BN, H), lambda i: (0, i, 0)),
            _row(H), _row(H), _full(H, OUT), _row(OUT),
        ],
        out_specs=pl.BlockSpec((BN, OUT), lambda i: (i, 0)),
        out_shape=jax.ShapeDtypeStruct((NP, OUT), jnp.float32),
    )(x, acca, accb, gx.reshape(1, H), bx.reshape(1, H), w,
      ob.reshape(1, OUT))


_SC_MESH = dict(core_axis_name="c", subcore_axis_name="s")


def _sc_gather(x_pad, comb):
    @functools.partial(
        pl.kernel,
        out_type=(jax.ShapeDtypeStruct((ES, H), jnp.float32),
                  jax.ShapeDtypeStruct((ES, H), jnp.float32)),
        mesh=plsc.VectorSubcoreMesh(**_SC_MESH),
        scratch_types=[
            pltpu.VMEM((K, 2 * C), jnp.int32),
            [pltpu.VMEM((2 * C, H), jnp.float32)] * 3,
            [pltpu.SemaphoreType.DMA] * 3,
            [pltpu.SemaphoreType.DMA] * 3,
            [pltpu.SemaphoreType.DMA] * 3,
            pltpu.VMEM_SHARED((NP, H), jnp.float32),
        ],
    )
    def k(x_hbm, comb_hbm, xi_hbm, xj_hbm, idx2, rb, g, od, os, shx):
        sid = lax.axis_index("s")
        wid = sid * NC + lax.axis_index("c")
        base = pl.multiple_of(wid * EW, EW)
        pltpu.sync_copy(comb_hbm.at[wid], idx2)
        tslc = pl.ds(pl.multiple_of(sid * RPT, RPT), RPT)
        pltpu.sync_copy(x_hbm.at[tslc], shx.at[tslc])
        plsc.subcore_barrier()

        def gath(j, b):
            pltpu.async_copy(shx.at[idx2.at[j]], rb[b], g[b])

        def wait_g(b):
            pltpu.make_async_copy(x_hbm.at[pl.ds(0, 2 * C)], rb[b], g[b]).wait()

        def outs(j, b):
            off = pl.multiple_of(base + j * C, C)
            pltpu.async_copy(rb[b].at[pl.ds(0, C)], xi_hbm.at[pl.ds(off, C)],
                             od[b])
            pltpu.async_copy(rb[b].at[pl.ds(C, C)], xj_hbm.at[pl.ds(off, C)],
                             os[b])

        def wait_outs(b):
            pltpu.make_async_copy(rb[b].at[pl.ds(0, C)],
                                  xi_hbm.at[pl.ds(base, C)], od[b]).wait()
            pltpu.make_async_copy(rb[b].at[pl.ds(C, C)],
                                  xj_hbm.at[pl.ds(base, C)], os[b]).wait()

        for b in range(3):
            gath(b, b)

        def body(i, carry):
            j0 = 3 * i
            for b in range(3):
                wait_g(b)
                outs(j0 + b, b)
            for b in range(3):
                wait_outs(b)
                gath(j0 + 3 + b, b)
            return carry

        lax.fori_loop(0, (K - 5) // 3, body, 0)
        for b in range(3):
            wait_g(b)
            outs(K - 5 + b, b)
        for b in range(2):
            wait_outs(b)
            gath(K - 2 + b, b)
        for b in range(2):
            wait_g(b)
            outs(K - 2 + b, b)
        for b in range(3):
            wait_outs(b)

    return k(x_pad, comb)


def _sc_scatter(m, dst3d, zrows):
    @functools.partial(
        pl.kernel,
        out_type=jax.ShapeDtypeStruct((NC, NP, H), jnp.float32),
        mesh=plsc.VectorSubcoreMesh(**_SC_MESH),
        scratch_types=[
            pltpu.VMEM((K, C), jnp.int32),
            [pltpu.VMEM((C, H), jnp.float32)] * 4,
            pltpu.VMEM_SHARED((NP, H), jnp.float32),
            [pltpu.SemaphoreType.DMA] * 4,
        ],
    )
    def k(m_hbm, dst_hbm, z_hbm, acc_hbm, idx2, rows, shacc, l):
        cid = lax.axis_index("c")
        sid = lax.axis_index("s")
        wid = sid * NC + cid
        base = pl.multiple_of(wid * EW, EW)
        pltpu.sync_copy(dst_hbm.at[wid], idx2)
        pltpu.sync_copy(z_hbm, shacc.at[pl.ds(pl.multiple_of(sid * RPT, RPT), RPT)])
        plsc.subcore_barrier()

        def load(j, b):
            pltpu.async_copy(m_hbm.at[pl.ds(base + j * C, C)], rows[b], l[b])

        def addchunk(j, b):
            pltpu.make_async_copy(m_hbm.at[pl.ds(0, C)], rows[b], l[b]).wait()
            pltpu.sync_copy(rows[b], shacc.at[idx2.at[j]], add=True)

        for b in range(4):
            load(b, b)

        def body(i, carry):
            j0 = 4 * i
            for b in range(4):
                addchunk(j0 + b, b)
                load(j0 + 4 + b, b)
            return carry

        lax.fori_loop(0, (K - 5) // 4, body, 0)
        addchunk(K - 5, 0)
        load(K - 1, 0)
        for b in range(1, 4):
            addchunk(K - 5 + b, b)
        addchunk(K - 1, 0)
        plsc.subcore_barrier()
        tslc = pl.ds(pl.multiple_of(sid * RPT, RPT), RPT)
        pltpu.sync_copy(shacc.at[tslc], acc_hbm.at[cid].at[tslc])

    return k(m, dst3d, zrows)


def kernel(x, edge_index, edge_attr, params):
    src = edge_index[0]
    dst = edge_index[1]
    dstr = dst.reshape(NSH, NW, K, C)
    srcr = src.reshape(NSH, NW, K, C)
    comb = jnp.concatenate([dstr, srcr], axis=-1)
    dst3d = [dstr[s] for s in range(NSH)]
    attrs = [lax.slice(edge_attr, (s * ES, 0), ((s + 1) * ES, ADJ))
             for s in range(NSH)]

    xpad3 = jnp.pad(x, (0, NP - N)).reshape(NP // BN, 1, BN)
    embp = jnp.zeros((VOCABP, H), jnp.float32).at[:params["emb"].shape[0]].set(
        params["emb"])
    zrows = jnp.zeros((RPT, H), jnp.float32)

    bn_scale = 1.0 / jnp.sqrt(1.0 + EPS)

    xc = _tc_node_init(xpad3, embp, params["ex_W"],
                       params["ex_b"].reshape(1, H),
                       params["ex_g"].reshape(1, H),
                       params["ex_beta"].reshape(1, H))

    ea = [None] * NSH
    for i in range(4):
        pc = params["conv"][i]
        ge = pc["bng_e"] * bn_scale
        gx = pc["bng_x"] * bn_scale
        acc = [None] * NSH
        gathered = [_sc_gather(xc, comb[s]) for s in range(NSH)]
        for s in range(NSH):
            xi, xj = gathered[s]
            if i == 0:
                m, ea[s] = _tc_edge_first(attrs[s], xi, xj, params, pc,
                                          ge, pc["bnb_e"])
            elif i < 3:
                m, ea[s] = _tc_edge_mid(ea[s], xi, xj, pc, ge, pc["bnb_e"])
            else:
                m = _tc_edge_last(ea[s], xi, xj, pc)
            acc[s] = _sc_scatter(m, dst3d[s], zrows)
        if i < 3:
            xc = _tc_node_mid(xc, acc[0], acc[1], gx, pc["bnb_x"])
        else:
            out = _tc_node_last(xc, acc[0], acc[1], gx, pc["bnb_x"],
                                params["out_W"], params["out_b"])
    return out[:N]


# speedup vs baseline: 1.0013x; 1.0013x over previous
import functools

import jax
import jax.numpy as jnp
from jax import lax
from jax.experimental import pallas as pl
from jax.experimental.pallas import tpu as pltpu
from jax.experimental.pallas import tpu_sc as plsc

EPS = 1e-5

NC = 2
NS = 16
NW = NC * NS

N = 10000
E = 320000
H = 128
ADJ = 16
OUT = 128
VOCABP = 32

NP = 10240
RPT = NP // NS
NSH = 2
ES = E // NSH
EW = ES // NW
C = 40
K = EW // C

BN = 1024
BE = 3200


def _node_init_body(idx_ref, emb_ref, w_ref, b_ref, g_ref, beta_ref, o_ref):
    idx = idx_ref[0]
    oht = (idx == lax.broadcasted_iota(jnp.int32, (VOCABP, BN), 0))
    e = lax.dot_general(oht.astype(jnp.float32), emb_ref[...],
                        (((0,), (0,)), ((), ())))
    e = jnp.maximum(e, 0.0)
    h = e @ w_ref[...] + b_ref[...]
    mu = jnp.mean(h, axis=-1, keepdims=True)
    var = jnp.mean((h - mu) * (h - mu), axis=-1, keepdims=True)
    o_ref[...] = (h - mu) * lax.rsqrt(var + EPS) * g_ref[...] + beta_ref[...]


def _edge_mlp(xi, xj, ea, w1_ref, b1_ref, w2_ref, b2_ref):
    h = (xi @ w1_ref[:H, :] + (xj - xi) @ w1_ref[H:2 * H, :]
         + ea @ w1_ref[2 * H:, :] + b1_ref[...])
    h = jnp.maximum(h, 0.0)
    return h @ w2_ref[...] + b2_ref[...]


def _edge_first_body(attr_ref, xi_ref, xj_ref,
                     ew1_ref, eb1_ref, ew2_ref, eb2_ref, eg_ref, ebeta_ref,
                     w1_ref, b1_ref, w2_ref, b2_ref, ge_ref, be_ref,
                     m_ref, eo_ref):
    t = jnp.maximum(attr_ref[...] @ ew1_ref[...] + eb1_ref[...], 0.0)
    t = t @ ew2_ref[...] + eb2_ref[...]
    mu = jnp.mean(t, axis=-1, keepdims=True)
    var = jnp.mean((t - mu) * (t - mu), axis=-1, keepdims=True)
    ea = (t - mu) * lax.rsqrt(var + EPS) * eg_ref[...] + ebeta_ref[...]
    m = _edge_mlp(xi_ref[...], xj_ref[...], ea, w1_ref, b1_ref, w2_ref, b2_ref)
    m_ref[...] = m
    eo_ref[...] = jnp.maximum(ea + m * ge_ref[...] + be_ref[...], 0.0)


def _edge_mid_body(ea_ref, xi_ref, xj_ref,
                   w1_ref, b1_ref, w2_ref, b2_ref, ge_ref, be_ref,
                   m_ref, eo_ref):
    ea = ea_ref[...]
    m = _edge_mlp(xi_ref[...], xj_ref[...], ea, w1_ref, b1_ref, w2_ref, b2_ref)
    m_ref[...] = m
    eo_ref[...] = jnp.maximum(ea + m * ge_ref[...] + be_ref[...], 0.0)


def _edge_last_body(ea_ref, xi_ref, xj_ref,
                    w1_ref, b1_ref, w2_ref, b2_ref,
                    m_ref):
    m_ref[...] = _edge_mlp(xi_ref[...], xj_ref[...], ea_ref[...],
                           w1_ref, b1_ref, w2_ref, b2_ref)


def _node_mid_body(x_ref, acca_ref, accb_ref, g_ref, b_ref, o_ref):
    s = (acca_ref[0] + acca_ref[1]) + (accb_ref[0] + accb_ref[1])
    o_ref[...] = jnp.maximum(x_ref[...] + s * g_ref[...] + b_ref[...], 0.0)


def _node_last_body(x_ref, acca_ref, accb_ref, g_ref, b_ref, w_ref, ob_ref,
                    o_ref):
    s = (acca_ref[0] + acca_ref[1]) + (accb_ref[0] + accb_ref[1])
    xn = x_ref[...] + s * g_ref[...] + b_ref[...]
    o_ref[...] = xn @ w_ref[...] + ob_ref[...]


def _row(n):
    return pl.BlockSpec((1, n), lambda i: (0, 0))


def _full(a, b):
    return pl.BlockSpec((a, b), lambda i: (0, 0))


def _tc_node_init(xpad3, embp, w, b, g, beta):
    return pl.pallas_call(
        _node_init_body,
        grid=(NP // BN,),
        in_specs=[
            pl.BlockSpec((1, 1, BN), lambda i: (i, 0, 0)),
            _full(VOCABP, H), _full(H, H), _row(H), _row(H), _row(H),
        ],
        out_specs=pl.BlockSpec((BN, H), lambda i: (i, 0)),
        out_shape=jax.ShapeDtypeStruct((NP, H), jnp.float32),
    )(xpad3, embp, w, b, g, beta)


def _tc_edge_first(attr, xi, xj, pe, pc, ge, be):
    blk = pl.BlockSpec((BE, H), lambda i: (i, 0))
    return pl.pallas_call(
        _edge_first_body,
        grid=(ES // BE,),
        in_specs=[
            pl.BlockSpec((BE, ADJ), lambda i: (i, 0)), blk, blk,
            _full(ADJ, H), _row(H), _full(H, H), _row(H), _row(H), _row(H),
            _full(3 * H, 2 * H), _row(2 * H), _full(2 * H, H), _row(H),
            _row(H), _row(H),
        ],
        out_specs=(blk, blk),
        out_shape=(jax.ShapeDtypeStruct((ES, H), jnp.float32),
                   jax.ShapeDtypeStruct((ES, H), jnp.float32)),
    )(attr, xi, xj,
      pe["ea_W1"], pe["ea_b1"].reshape(1, H), pe["ea_W2"],
      pe["ea_b2"].reshape(1, H), pe["ea_g"].reshape(1, H),
      pe["ea_beta"].reshape(1, H),
      pc["W1"], pc["b1"].reshape(1, 2 * H), pc["W2"], pc["b2"].reshape(1, H),
      ge.reshape(1, H), be.reshape(1, H))


def _tc_edge_mid(ea, xi, xj, pc, ge, be):
    blk = pl.BlockSpec((BE, H), lambda i: (i, 0))
    return pl.pallas_call(
        _edge_mid_body,
        grid=(ES // BE,),
        in_specs=[
            blk, blk, blk,
            _full(3 * H, 2 * H), _row(2 * H), _full(2 * H, H), _row(H),
            _row(H), _row(H),
        ],
        out_specs=(blk, blk),
        out_shape=(jax.ShapeDtypeStruct((ES, H), jnp.float32),
                   jax.ShapeDtypeStruct((ES, H), jnp.float32)),
    )(ea, xi, xj, pc["W1"], pc["b1"].reshape(1, 2 * H), pc["W2"],
      pc["b2"].reshape(1, H), ge.reshape(1, H), be.reshape(1, H))


def _tc_edge_last(ea, xi, xj, pc):
    blk = pl.BlockSpec((BE, H), lambda i: (i, 0))
    return pl.pallas_call(
        _edge_last_body,
        grid=(ES // BE,),
        in_specs=[
            blk, blk, blk,
            _full(3 * H, 2 * H), _row(2 * H), _full(2 * H, H), _row(H),
        ],
        out_specs=blk,
        out_shape=jax.ShapeDtypeStruct((ES, H), jnp.float32),
    )(ea, xi, xj, pc["W1"], pc["b1"].reshape(1, 2 * H), pc["W2"],
      pc["b2"].reshape(1, H))


def _tc_node_mid(x, acca, accb, gx, bx):
    return pl.pallas_call(
        _node_mid_body,
        grid=(NP // BN,),
        in_specs=[
            pl.BlockSpec((BN, H), lambda i: (i, 0)),
            pl.BlockSpec((2, BN, H), lambda i: (0, i, 0)),
            pl.BlockSpec((2, BN, H), lambda i: (0, i, 0)),
            _row(H), _row(H),
        ],
        out_specs=pl.BlockSpec((BN, H), lambda i: (i, 0)),
        out_shape=jax.ShapeDtypeStruct((NP, H), jnp.float32),
    )(x, acca, accb, gx.reshape(1, H), bx.reshape(1, H))


def _tc_node_last(x, acca, accb, gx, bx, w, ob):
    return pl.pallas_call(
        _node_last_body,
        grid=(NP // BN,),
        in_specs=[
            pl.BlockSpec((BN, H), lambda i: (i, 0)),
            pl.BlockSpec((2, BN, H), lambda i: (0, i, 0)),
            pl.BlockSpec((2, BN, H), lambda i: (0, i, 0)),
            _row(H), _row(H), _full(H, OUT), _row(OUT),
        ],
        out_specs=pl.BlockSpec((BN, OUT), lambda i: (i, 0)),
        out_shape=jax.ShapeDtypeStruct((NP, OUT), jnp.float32),
    )(x, acca, accb, gx.reshape(1, H), bx.reshape(1, H), w,
      ob.reshape(1, OUT))


_SC_MESH = dict(core_axis_name="c", subcore_axis_name="s")


def _sc_gather(x_pad, comb):
    @functools.partial(
        pl.kernel,
        out_type=(jax.ShapeDtypeStruct((ES, H), jnp.float32),
                  jax.ShapeDtypeStruct((ES, H), jnp.float32)),
        mesh=plsc.VectorSubcoreMesh(**_SC_MESH),
        scratch_types=[
            pltpu.VMEM((K, 2 * C), jnp.int32),
            [pltpu.VMEM((2 * C, H), jnp.float32)] * 3,
            [pltpu.SemaphoreType.DMA] * 3,
            [pltpu.SemaphoreType.DMA] * 3,
            [pltpu.SemaphoreType.DMA] * 3,
            pltpu.VMEM_SHARED((NP, H), jnp.float32),
        ],
    )
    def k(x_hbm, comb_hbm, xi_hbm, xj_hbm, idx2, rb, g, od, os, shx):
        sid = lax.axis_index("s")
        wid = sid * NC + lax.axis_index("c")
        base = pl.multiple_of(wid * EW, EW)
        pltpu.sync_copy(comb_hbm.at[wid], idx2)
        tslc = pl.ds(pl.multiple_of(sid * RPT, RPT), RPT)
        pltpu.sync_copy(x_hbm.at[tslc], shx.at[tslc])
        plsc.subcore_barrier()

        def gath(j, b):
            pltpu.async_copy(shx.at[idx2.at[j]], rb[b], g[b])

        def wait_g(b):
            pltpu.make_async_copy(x_hbm.at[pl.ds(0, 2 * C)], rb[b], g[b]).wait()

        def outs(j, b):
            off = pl.multiple_of(base + j * C, C)
            pltpu.async_copy(rb[b].at[pl.ds(0, C)], xi_hbm.at[pl.ds(off, C)],
                             od[b])
            pltpu.async_copy(rb[b].at[pl.ds(C, C)], xj_hbm.at[pl.ds(off, C)],
                             os[b])

        def wait_outs(b):
            pltpu.make_async_copy(rb[b].at[pl.ds(0, C)],
                                  xi_hbm.at[pl.ds(base, C)], od[b]).wait()
            pltpu.make_async_copy(rb[b].at[pl.ds(C, C)],
                                  xj_hbm.at[pl.ds(base, C)], os[b]).wait()

        for b in range(3):
            gath(b, b)

        def body(i, carry):
            j0 = 3 * i
            for b in range(3):
                wait_g(b)
                outs(j0 + b, b)
            for b in range(3):
                wait_outs(b)
                gath(j0 + 3 + b, b)
            return carry

        lax.fori_loop(0, (K - 5) // 3, body, 0)
        for b in range(3):
            wait_g(b)
            outs(K - 5 + b, b)
        for b in range(2):
            wait_outs(b)
            gath(K - 2 + b, b)
        for b in range(2):
            wait_g(b)
            outs(K - 2 + b, b)
        for b in range(3):
            wait_outs(b)

    return k(x_pad, comb)


def _sc_scatter(m, dst3d, zrows):
    @functools.partial(
        pl.kernel,
        out_type=jax.ShapeDtypeStruct((NC, NP, H), jnp.float32),
        mesh=plsc.VectorSubcoreMesh(**_SC_MESH),
        scratch_types=[
            pltpu.VMEM((K, C), jnp.int32),
            [pltpu.VMEM((C, H), jnp.float32)] * 4,
            pltpu.VMEM_SHARED((NP, H), jnp.float32),
            [pltpu.SemaphoreType.DMA] * 4,
        ],
    )
    def k(m_hbm, dst_hbm, z_hbm, acc_hbm, idx2, rows, shacc, l):
        cid = lax.axis_index("c")
        sid = lax.axis_index("s")
        wid = sid * NC + cid
        base = pl.multiple_of(wid * EW, EW)
        pltpu.sync_copy(dst_hbm.at[wid], idx2)
        pltpu.sync_copy(z_hbm, shacc.at[pl.ds(pl.multiple_of(sid * RPT, RPT), RPT)])
        plsc.subcore_barrier()

        def load(j, b):
            pltpu.async_copy(m_hbm.at[pl.ds(base + j * C, C)], rows[b], l[b])

        def addchunk(j, b):
            pltpu.make_async_copy(m_hbm.at[pl.ds(0, C)], rows[b], l[b]).wait()
            pltpu.sync_copy(rows[b], shacc.at[idx2.at[j]], add=True)

        for b in range(4):
            load(b, b)

        def body(i, carry):
            j0 = 4 * i
            for b in range(4):
                addchunk(j0 + b, b)
                load(j0 + 4 + b, b)
            return carry

        lax.fori_loop(0, (K - 5) // 4, body, 0)
        addchunk(K - 5, 0)
        load(K - 1, 0)
        for b in range(1, 4):
            addchunk(K - 5 + b, b)
        addchunk(K - 1, 0)
        plsc.subcore_barrier()
        tslc = pl.ds(pl.multiple_of(sid * RPT, RPT), RPT)
        pltpu.sync_copy(shacc.at[tslc], acc_hbm.at[cid].at[tslc])

    return k(m, dst3d, zrows)


def kernel(x, edge_index, edge_attr, params):
    src = edge_index[0]
    dst = edge_index[1]
    dstr = dst.reshape(NSH, NW, K, C)
    srcr = src.reshape(NSH, NW, K, C)
    comb = jnp.concatenate([dstr, srcr], axis=-1)
    dst3d = [dstr[s] for s in range(NSH)]
    attrs = [lax.slice(edge_attr, (s * ES, 0), ((s + 1) * ES, ADJ))
             for s in range(NSH)]

    xpad3 = jnp.pad(x, (0, NP - N)).reshape(NP // BN, 1, BN)
    embp = jnp.zeros((VOCABP, H), jnp.float32).at[:params["emb"].shape[0]].set(
        params["emb"])
    zrows = jnp.zeros((RPT, H), jnp.float32)

    bn_scale = 1.0 / jnp.sqrt(1.0 + EPS)

    xc = _tc_node_init(xpad3, embp, params["ex_W"],
                       params["ex_b"].reshape(1, H),
                       params["ex_g"].reshape(1, H),
                       params["ex_beta"].reshape(1, H))

    ea = [None] * NSH
    for i in range(4):
        pc = params["conv"][i]
        ge = pc["bng_e"] * bn_scale
        gx = pc["bng_x"] * bn_scale
        acc = [None] * NSH
        gathered = [_sc_gather(xc, comb[s]) for s in range(NSH)]
        for s in range(NSH):
            xi, xj = gathered[s]
            if i == 0:
                m, ea[s] = _tc_edge_first(attrs[s], xi, xj, params, pc,
                                          ge, pc["bnb_e"])
            elif i < 3:
                m, ea[s] = _tc_edge_mid(ea[s], xi, xj, pc, ge, pc["bnb_e"])
            else:
                m = _tc_edge_last(ea[s], xi, xj, pc)
            acc[s] = _sc_scatter(m, dst3d[s], zrows)
        if i < 3:
            xc = _tc_node_mid(xc, acc[0], acc[1], gx, pc["bnb_x"])
        else:
            out = _tc_node_last(xc, acc[0], acc[1], gx, pc["bnb_x"],
                                params["out_W"], params["out_b"])
    return out[:N]
